# baseline (speedup 1.0000x reference)
import functools

import jax
import jax.numpy as jnp
from jax import lax
from jax.experimental import pallas as pl
from jax.experimental.pallas import tpu as pltpu
from jax.experimental.pallas import tpu_sc as plsc

CAP_FACTOR = 2.0


def _router_body(x_ref, wr_ref, aff_ref):
    x = x_ref[0]
    wr = wr_ref[...]
    logits = jax.lax.dot_general(wr, x, (((1,), (1,)), ((), ())),
                                 preferred_element_type=jnp.float32)
    m = jnp.max(logits, axis=0, keepdims=True)
    p = jnp.exp(logits - m)
    aff_ref[0] = p / jnp.sum(p, axis=0, keepdims=True)


def _ffn_body(x_ref, w1_ref, b1_ref, w2_ref, b2_ref, gate_ref, out_ref):
    h_idx = pl.program_id(2)
    x = x_ref[0, 0]
    w1 = w1_ref[0].astype(jnp.bfloat16)
    b1 = b1_ref[0]
    h = jax.lax.dot_general(x, w1, (((1,), (1,)), ((), ())),
                            preferred_element_type=jnp.float32)
    h = h + b1
    h = 0.5 * h * (1.0 + jax.lax.erf(h * 0.7071067811865476))
    gate = gate_ref[0, 0]
    h = (h * gate.reshape(-1, 1)).astype(jnp.bfloat16)
    w2 = w2_ref[0].astype(jnp.bfloat16)
    y = jax.lax.dot_general(h, w2, (((1,), (1,)), ((), ())),
                            preferred_element_type=jnp.float32)

    @pl.when(h_idx == 0)
    def _():
        out_ref[0, 0] = y + gate.reshape(-1, 1) * b2_ref[0]

    @pl.when(h_idx > 0)
    def _():
        out_ref[0, 0] += y


def _make_routing_sc(n_rows, S, cap, capp):
    n_chunks = S // 16
    mesh = plsc.VectorSubcoreMesh(core_axis_name="c", subcore_axis_name="s")

    @functools.partial(
        pl.kernel,
        out_type=(jax.ShapeDtypeStruct((n_rows * capp,), jnp.int32),
                  jax.ShapeDtypeStruct((n_rows * capp,), jnp.float32)),
        mesh=mesh,
        scratch_types=[
            pltpu.VMEM((S,), jnp.float32),
            pltpu.VMEM((capp,), jnp.int32),
            pltpu.VMEM((capp,), jnp.float32),
        ],
        compiler_params=pltpu.CompilerParams(needs_layout_passes=False),
    )
    def routing(aff, idx_out, gate_out, row_v, idx_v, gate_v):
        wid = lax.axis_index("s") * 2 + lax.axis_index("c")
        pltpu.sync_copy(aff.at[wid], row_v)

        def initbody(i, _):
            idx_v[pl.ds(i * 16, 16)] = jnp.zeros((16,), jnp.int32)
            gate_v[pl.ds(i * 16, 16)] = jnp.zeros((16,), jnp.float32)
            return 0
        lax.fori_loop(0, capp // 16, initbody, 0)

        ones = jnp.ones((16,), jnp.int32)
        zeros = jnp.zeros((16,), jnp.int32)

        def quant(v):
            return plsc.bitcast(v, jnp.int32)

        def count_ge(t):
            def cbody(i, acc):
                q = quant(row_v[pl.ds(i * 16, 16)])
                return acc + jnp.where(q >= t, ones, zeros)
            acc = lax.fori_loop(0, n_chunks, cbody,
                                jnp.zeros((16,), jnp.int32))
            return jnp.sum(acc)

        def bsbody(i, lohi):
            lo, hi = lohi
            mid = lo + ((hi - lo) >> 1)
            ok = count_ge(mid) >= cap
            return (jnp.where(ok, mid, lo), jnp.where(ok, hi, mid))
        lo, _ = lax.fori_loop(0, 31, bsbody,
                              (jnp.int32(0), jnp.int32(0x42000000)))
        thr = lo
        n_gt = count_ge(thr + 1)
        extra = cap - n_gt

        def selbody(i, carry):
            off, eq_taken = carry
            v = row_v[pl.ds(i * 16, 16)]
            q = quant(v)
            m_gt = q > thr
            m_eq = q == thr
            eq_rank = plsc.cumsum(jnp.where(m_eq, ones, zeros)) + eq_taken
            m = m_gt | (m_eq & (eq_rank <= extra))
            mi = jnp.where(m, ones, zeros)
            pos = off + plsc.cumsum(mi) - 1
            ids = i * 16 + lax.iota(jnp.int32, 16)
            plsc.store_scatter(idx_v, [pos], ids, mask=m)
            plsc.store_scatter(gate_v, [pos], v, mask=m)
            return (off + jnp.sum(mi),
                    eq_taken + jnp.sum(jnp.where(m & m_eq, ones, zeros)))
        lax.fori_loop(0, n_chunks, selbody, (jnp.int32(0), jnp.int32(0)))

        obase = pl.multiple_of(wid * capp, 8)
        pltpu.sync_copy(idx_v, idx_out.at[pl.ds(obase, capp)])
        pltpu.sync_copy(gate_v, gate_out.at[pl.ds(obase, capp)])

    return routing


def _combine_body(idx_ref, yg_ref, out_ref):
    e_idx = pl.program_id(1)
    S = out_ref.shape[1]
    idx = idx_ref[0, 0]
    yg = yg_ref[0, 0].astype(jnp.bfloat16)
    toks = jax.lax.broadcasted_iota(jnp.int32, (idx.shape[1], S), 1)
    P = (idx.reshape(-1, 1) == toks).astype(jnp.bfloat16)
    y = jax.lax.dot_general(P, yg, (((0,), (0,)), ((), ())),
                            preferred_element_type=jnp.float32)

    @pl.when(e_idx == 0)
    def _():
        out_ref[0] = y

    @pl.when(e_idx > 0)
    def _():
        out_ref[0] += y


def kernel(x_prime, Wr, fc1_w, fc1_b, fc2_w, fc2_b):
    B, S, D = x_prime.shape
    E, H, _ = fc1_w.shape
    cap = max(1, int(S * CAP_FACTOR / E) + 1)
    capp = -(-cap // 16) * 16

    St = min(512, S)
    aff_t = pl.pallas_call(
        _router_body,
        grid=(B, S // St),
        in_specs=[
            pl.BlockSpec((1, St, D), lambda b, s: (b, s, 0)),
            pl.BlockSpec((E, D), lambda b, s: (0, 0)),
        ],
        out_specs=pl.BlockSpec((1, E, St), lambda b, s: (b, 0, s)),
        out_shape=jax.ShapeDtypeStruct((B, E, S), jnp.float32),
    )(x_prime, Wr)

    routing = _make_routing_sc(B * E, S, cap, capp)
    idxl2d, gatep = routing(aff_t.reshape(B * E, S))
    idxl = idxl2d.reshape(B, E, capp)
    gatep = gatep.reshape(B, E, capp)

    x_bf = x_prime.astype(jnp.bfloat16)
    xg = x_bf[jnp.arange(B)[:, None, None], idxl]

    Ht = min(1024, H)
    yg = pl.pallas_call(
        _ffn_body,
        grid=(B, E, H // Ht),
        in_specs=[
            pl.BlockSpec((1, 1, capp, D), lambda b, e, h: (b, e, 0, 0)),
            pl.BlockSpec((1, Ht, D), lambda b, e, h: (e, h, 0)),
            pl.BlockSpec((1, 1, Ht), lambda b, e, h: (e, 0, h)),
            pl.BlockSpec((1, D, Ht), lambda b, e, h: (e, 0, h)),
            pl.BlockSpec((1, 1, D), lambda b, e, h: (e, 0, 0)),
            pl.BlockSpec((1, 1, 1, capp), lambda b, e, h: (b, e, 0, 0)),
        ],
        out_specs=pl.BlockSpec((1, 1, capp, D), lambda b, e, h: (b, e, 0, 0)),
        out_shape=jax.ShapeDtypeStruct((B, E, capp, D), jnp.float32),
        compiler_params=pltpu.CompilerParams(
            dimension_semantics=("parallel", "parallel", "arbitrary")),
    )(xg, fc1_w, fc1_b.reshape(E, 1, H), fc2_w, fc2_b.reshape(E, 1, D),
      gatep.reshape(B, E, 1, capp))

    out = pl.pallas_call(
        _combine_body,
        grid=(B, E),
        in_specs=[
            pl.BlockSpec((1, 1, 1, capp), lambda b, e: (b, e, 0, 0)),
            pl.BlockSpec((1, 1, capp, D), lambda b, e: (b, e, 0, 0)),
        ],
        out_specs=pl.BlockSpec((1, S, D), lambda b, e: (b, 0, 0)),
        out_shape=jax.ShapeDtypeStruct((B, S, D), jnp.float32),
        compiler_params=pltpu.CompilerParams(
            dimension_semantics=("parallel", "arbitrary")),
    )(idxl.reshape(B, E, 1, capp), yg)
    return out


# speedup vs baseline: 1.2963x; 1.2963x over previous
import functools

import jax
import jax.numpy as jnp
from jax import lax
from jax.experimental import pallas as pl
from jax.experimental.pallas import tpu as pltpu
from jax.experimental.pallas import tpu_sc as plsc

CAP_FACTOR = 2.0


def _router_body(x_ref, wr_ref, aff_ref):
    x = x_ref[0]
    wr = wr_ref[...]
    logits = jax.lax.dot_general(wr, x, (((1,), (1,)), ((), ())),
                                 preferred_element_type=jnp.float32)
    m = jnp.max(logits, axis=0, keepdims=True)
    p = jnp.exp(logits - m)
    aff_ref[0] = p / jnp.sum(p, axis=0, keepdims=True)


def _ffn_body(x_ref, w1_ref, b1_ref, w2_ref, b2_ref, gate_ref, out_ref,
              acc_ref):
    h_idx = pl.program_id(2)
    n_h = pl.num_programs(2)
    x = x_ref[0, 0].astype(jnp.bfloat16)
    w1 = w1_ref[0].astype(jnp.bfloat16)
    b1 = b1_ref[0]
    h = jax.lax.dot_general(x, w1, (((1,), (1,)), ((), ())),
                            preferred_element_type=jnp.float32)
    h = h + b1
    h = 0.5 * h * (1.0 + jax.lax.erf(h * 0.7071067811865476))
    gate = gate_ref[0, 0]
    h = (h * gate.reshape(-1, 1)).astype(jnp.bfloat16)
    w2 = w2_ref[0].astype(jnp.bfloat16)
    y = jax.lax.dot_general(h, w2, (((1,), (1,)), ((), ())),
                            preferred_element_type=jnp.float32)

    @pl.when(h_idx == 0)
    def _():
        acc_ref[...] = y + gate.reshape(-1, 1) * b2_ref[0]

    @pl.when(h_idx > 0)
    def _():
        acc_ref[...] += y

    @pl.when(h_idx == n_h - 1)
    def _():
        out_ref[0, 0] = acc_ref[...].astype(jnp.bfloat16)


def _make_routing_sc(n_rows, S, cap, capp):
    n_chunks = S // 16
    mesh = plsc.VectorSubcoreMesh(core_axis_name="c", subcore_axis_name="s")

    @functools.partial(
        pl.kernel,
        out_type=(jax.ShapeDtypeStruct((n_rows * capp,), jnp.int32),
                  jax.ShapeDtypeStruct((n_rows * capp,), jnp.float32)),
        mesh=mesh,
        scratch_types=[
            pltpu.VMEM((S,), jnp.float32),
            pltpu.VMEM((capp,), jnp.int32),
            pltpu.VMEM((capp,), jnp.float32),
        ],
        compiler_params=pltpu.CompilerParams(needs_layout_passes=False),
    )
    def routing(aff, idx_out, gate_out, row_v, idx_v, gate_v):
        wid = lax.axis_index("s") * 2 + lax.axis_index("c")
        pltpu.sync_copy(aff.at[wid], row_v)

        def initbody(i, _):
            idx_v[pl.ds(i * 16, 16)] = jnp.zeros((16,), jnp.int32)
            gate_v[pl.ds(i * 16, 16)] = jnp.zeros((16,), jnp.float32)
            return 0
        lax.fori_loop(0, capp // 16, initbody, 0)

        ones = jnp.ones((16,), jnp.int32)
        zeros = jnp.zeros((16,), jnp.int32)

        def quant(v):
            return plsc.bitcast(v, jnp.int32)

        def count_ge(t):
            def cbody(i, acc):
                q = quant(row_v[pl.ds(i * 16, 16)])
                return acc + jnp.where(q >= t, ones, zeros)
            acc = lax.fori_loop(0, n_chunks, cbody,
                                jnp.zeros((16,), jnp.int32))
            return jnp.sum(acc)

        def bsbody(i, lohi):
            lo, hi = lohi
            mid = lo + ((hi - lo) >> 1)
            ok = count_ge(mid) >= cap
            return (jnp.where(ok, mid, lo), jnp.where(ok, hi, mid))
        lo, _ = lax.fori_loop(0, 31, bsbody,
                              (jnp.int32(0), jnp.int32(0x42000000)))
        thr = lo
        n_gt = count_ge(thr + 1)
        extra = cap - n_gt

        def selbody(i, carry):
            off, eq_taken = carry
            v = row_v[pl.ds(i * 16, 16)]
            q = quant(v)
            m_gt = q > thr
            m_eq = q == thr
            eq_rank = plsc.cumsum(jnp.where(m_eq, ones, zeros)) + eq_taken
            m = m_gt | (m_eq & (eq_rank <= extra))
            mi = jnp.where(m, ones, zeros)
            pos = off + plsc.cumsum(mi) - 1
            ids = i * 16 + lax.iota(jnp.int32, 16)
            plsc.store_scatter(idx_v, [pos], ids, mask=m)
            plsc.store_scatter(gate_v, [pos], v, mask=m)
            return (off + jnp.sum(mi),
                    eq_taken + jnp.sum(jnp.where(m & m_eq, ones, zeros)))
        lax.fori_loop(0, n_chunks, selbody, (jnp.int32(0), jnp.int32(0)))

        obase = pl.multiple_of(wid * capp, 8)
        pltpu.sync_copy(idx_v, idx_out.at[pl.ds(obase, capp)])
        pltpu.sync_copy(gate_v, gate_out.at[pl.ds(obase, capp)])

    return routing


def _combine_body(idx_ref, yg_ref, out_ref):
    e_idx = pl.program_id(1)
    S = out_ref.shape[1]
    idx = idx_ref[0, 0]
    yg = yg_ref[0, 0]
    toks = jax.lax.broadcasted_iota(jnp.int32, (idx.shape[1], S), 1)
    P = (idx.reshape(-1, 1) == toks).astype(jnp.bfloat16)
    y = jax.lax.dot_general(P, yg, (((0,), (0,)), ((), ())),
                            preferred_element_type=jnp.float32)

    @pl.when(e_idx == 0)
    def _():
        out_ref[0] = y

    @pl.when(e_idx > 0)
    def _():
        out_ref[0] += y


def kernel(x_prime, Wr, fc1_w, fc1_b, fc2_w, fc2_b):
    B, S, D = x_prime.shape
    E, H, _ = fc1_w.shape
    cap = max(1, int(S * CAP_FACTOR / E) + 1)
    capp = -(-cap // 16) * 16

    St = min(512, S)
    aff_t = pl.pallas_call(
        _router_body,
        grid=(B, S // St),
        in_specs=[
            pl.BlockSpec((1, St, D), lambda b, s: (b, s, 0)),
            pl.BlockSpec((E, D), lambda b, s: (0, 0)),
        ],
        out_specs=pl.BlockSpec((1, E, St), lambda b, s: (b, 0, s)),
        out_shape=jax.ShapeDtypeStruct((B, E, S), jnp.float32),
    )(x_prime, Wr)

    routing = _make_routing_sc(B * E, S, cap, capp)
    idxl2d, gatep = routing(aff_t.reshape(B * E, S))
    idxl = idxl2d.reshape(B, E, capp)
    gatep = gatep.reshape(B, E, capp)

    xg = x_prime[jnp.arange(B)[:, None, None], idxl]

    Ht = min(1024, H)
    yg = pl.pallas_call(
        _ffn_body,
        grid=(B, E, H // Ht),
        in_specs=[
            pl.BlockSpec((1, 1, capp, D), lambda b, e, h: (b, e, 0, 0)),
            pl.BlockSpec((1, Ht, D), lambda b, e, h: (e, h, 0)),
            pl.BlockSpec((1, 1, Ht), lambda b, e, h: (e, 0, h)),
            pl.BlockSpec((1, D, Ht), lambda b, e, h: (e, 0, h)),
            pl.BlockSpec((1, 1, D), lambda b, e, h: (e, 0, 0)),
            pl.BlockSpec((1, 1, 1, capp), lambda b, e, h: (b, e, 0, 0)),
        ],
        out_specs=pl.BlockSpec((1, 1, capp, D), lambda b, e, h: (b, e, 0, 0)),
        out_shape=jax.ShapeDtypeStruct((B, E, capp, D), jnp.bfloat16),
        scratch_shapes=[pltpu.VMEM((capp, D), jnp.float32)],
        compiler_params=pltpu.CompilerParams(
            dimension_semantics=("parallel", "parallel", "arbitrary")),
    )(xg, fc1_w, fc1_b.reshape(E, 1, H), fc2_w, fc2_b.reshape(E, 1, D),
      gatep.reshape(B, E, 1, capp))

    out = pl.pallas_call(
        _combine_body,
        grid=(B, E),
        in_specs=[
            pl.BlockSpec((1, 1, 1, capp), lambda b, e: (b, e, 0, 0)),
            pl.BlockSpec((1, 1, capp, D), lambda b, e: (b, e, 0, 0)),
        ],
        out_specs=pl.BlockSpec((1, S, D), lambda b, e: (b, 0, 0)),
        out_shape=jax.ShapeDtypeStruct((B, S, D), jnp.float32),
        compiler_params=pltpu.CompilerParams(
            dimension_semantics=("parallel", "arbitrary")),
    )(idxl.reshape(B, E, 1, capp), yg)
    return out
